# baseline (speedup 1.0000x reference)
import jax
import jax.numpy as jnp
import numpy as np
from jax.experimental import pallas as pl
from jax.experimental.pallas import tpu as pltpu
from jax.experimental.pallas import tpu_sc as plsc

_PERCENTAGE = 0.5
_SEED = 0
_B = 16

def _subset_rows():
    try:
        cpu = jax.devices("cpu")[0]
        with jax.default_device(cpu):
            perm = np.asarray(jax.random.permutation(jax.random.key(_SEED), _B))
        sel = sorted(int(v) for v in perm[: int(_B * _PERCENTAGE)])
    except Exception:
        sel = [0, 1, 4, 5, 6, 8, 12, 13]
    return tuple(sel)

_SEL = _subset_rows()

_R2 = 4704
_C2 = 1024
_DMA_BLOCK = (8, _C2)
_REG = (1, 16)


def _relu_body(in_vmem, out_vmem):
    @pl.loop(0, _DMA_BLOCK[0], step=_REG[0])
    def _(c0):
        @pl.loop(0, _DMA_BLOCK[1], step=_REG[1])
        def _(c1):
            slc = (pl.ds(c0, _REG[0]), pl.ds(c1, _REG[1]))
            out_vmem.at[*slc][...] = jnp.maximum(in_vmem.at[*slc][...], 0.0)


def _copy_body(in_vmem, out_vmem):
    @pl.loop(0, _DMA_BLOCK[0], step=_REG[0])
    def _(c0):
        @pl.loop(0, _DMA_BLOCK[1], step=_REG[1])
        def _(c1):
            slc = (pl.ds(c0, _REG[0]), pl.ds(c1, _REG[1]))
            out_vmem.at[*slc][...] = in_vmem.at[*slc][...]


def kernel(x):
    xv = x.reshape(_B, _R2, _C2)
    mesh = plsc.VectorSubcoreMesh(core_axis_name="c", subcore_axis_name="s")

    @pl.kernel(out_type=jax.ShapeDtypeStruct((_B, _R2, _C2), jnp.float32),
               mesh=mesh)
    def sc_masked_relu(x_hbm, o_hbm):
        for row in range(_B):
            body = _relu_body if row in _SEL else _copy_body
            pltpu.emit_pipeline(
                body,
                grid=(_R2 // _DMA_BLOCK[0],),
                in_specs=[pl.BlockSpec(block_shape=_DMA_BLOCK,
                                       index_map=lambda i: (i, 0))],
                out_specs=[pl.BlockSpec(block_shape=_DMA_BLOCK,
                                        index_map=lambda i: (i, 0))],
                core_axis_name=("c", "s"),
                dimension_semantics=(pltpu.PARALLEL,),
            )(x_hbm.at[row], o_hbm.at[row])

    return sc_masked_relu(xv).reshape(x.shape)


# speedup vs baseline: 1.7715x; 1.7715x over previous
import jax
import jax.numpy as jnp
import numpy as np
from jax.experimental import pallas as pl
from jax.experimental.pallas import tpu as pltpu

_PERCENTAGE = 0.5
_SEED = 0
_B = 16

def _subset_rows():
    try:
        cpu = jax.devices("cpu")[0]
        with jax.default_device(cpu):
            perm = np.asarray(jax.random.permutation(jax.random.key(_SEED), _B))
        sel = sorted(int(v) for v in perm[: int(_B * _PERCENTAGE)])
    except Exception:
        sel = [0, 1, 4, 5, 6, 8, 12, 13]
    return tuple(sel)

_SEL = _subset_rows()
_MASK = tuple(1 if b in _SEL else 0 for b in range(_B))

_R = 96
_C = 224 * 224
_RB = 16


def _masked_relu_body(mask_ref, x_ref, o_ref):
    b = pl.program_id(0)
    sel = mask_ref[b] != 0

    @pl.when(sel)
    def _():
        o_ref[...] = jnp.maximum(x_ref[...], 0.0)

    @pl.when(jnp.logical_not(sel))
    def _():
        o_ref[...] = x_ref[...]


def kernel(x):
    mask = jnp.asarray(_MASK, dtype=jnp.int32)
    xv = x.reshape(_B, _R, _C)
    out = pl.pallas_call(
        _masked_relu_body,
        grid_spec=pltpu.PrefetchScalarGridSpec(
            num_scalar_prefetch=1,
            grid=(_B, _R // _RB),
            in_specs=[pl.BlockSpec((1, _RB, _C), lambda b, r, m: (b, r, 0))],
            out_specs=pl.BlockSpec((1, _RB, _C), lambda b, r, m: (b, r, 0)),
        ),
        out_shape=jax.ShapeDtypeStruct((_B, _R, _C), x.dtype),
    )(mask, xv)
    return out.reshape(x.shape)


# speedup vs baseline: 1.7738x; 1.0013x over previous
import jax
import jax.numpy as jnp
import numpy as np
from jax.experimental import pallas as pl
from jax.experimental.pallas import tpu as pltpu

_PERCENTAGE = 0.5
_SEED = 0
_B = 16

def _subset_rows():
    try:
        cpu = jax.devices("cpu")[0]
        with jax.default_device(cpu):
            perm = np.asarray(jax.random.permutation(jax.random.key(_SEED), _B))
        sel = sorted(int(v) for v in perm[: int(_B * _PERCENTAGE)])
    except Exception:
        sel = [0, 1, 4, 5, 6, 8, 12, 13]
    return tuple(sel)

_SEL = _subset_rows()
_MASK = tuple(1 if b in _SEL else 0 for b in range(_B))

_R = 96
_C = 224 * 224
_RB = 32


def _masked_relu_body(mask_ref, x_ref, o_ref):
    b = pl.program_id(0)
    sel = mask_ref[b] != 0

    @pl.when(sel)
    def _():
        o_ref[...] = jnp.maximum(x_ref[...], 0.0)

    @pl.when(jnp.logical_not(sel))
    def _():
        o_ref[...] = x_ref[...]


def kernel(x):
    mask = jnp.asarray(_MASK, dtype=jnp.int32)
    xv = x.reshape(_B, _R, _C)
    out = pl.pallas_call(
        _masked_relu_body,
        grid_spec=pltpu.PrefetchScalarGridSpec(
            num_scalar_prefetch=1,
            grid=(_B, _R // _RB),
            in_specs=[pl.BlockSpec((1, _RB, _C), lambda b, r, m: (b, r, 0))],
            out_specs=pl.BlockSpec((1, _RB, _C), lambda b, r, m: (b, r, 0)),
        ),
        out_shape=jax.ShapeDtypeStruct((_B, _R, _C), x.dtype),
    )(mask, xv)
    return out.reshape(x.shape)


# speedup vs baseline: 1.7763x; 1.0014x over previous
import jax
import jax.numpy as jnp
import numpy as np
from jax.experimental import pallas as pl
from jax.experimental.pallas import tpu as pltpu

_PERCENTAGE = 0.5
_SEED = 0
_B = 16

def _subset_rows():
    try:
        cpu = jax.devices("cpu")[0]
        with jax.default_device(cpu):
            perm = np.asarray(jax.random.permutation(jax.random.key(_SEED), _B))
        sel = sorted(int(v) for v in perm[: int(_B * _PERCENTAGE)])
    except Exception:
        sel = [0, 1, 4, 5, 6, 8, 12, 13]
    return tuple(sel)

_SEL = _subset_rows()
_MASK = tuple(1 if b in _SEL else 0 for b in range(_B))

_R = 96
_C = 224 * 224
_RB = 48


def _masked_relu_body(mask_ref, x_ref, o_ref):
    b = pl.program_id(0)
    sel = mask_ref[b] != 0

    @pl.when(sel)
    def _():
        o_ref[...] = jnp.maximum(x_ref[...], 0.0)

    @pl.when(jnp.logical_not(sel))
    def _():
        o_ref[...] = x_ref[...]


def kernel(x):
    mask = jnp.asarray(_MASK, dtype=jnp.int32)
    xv = x.reshape(_B, _R, _C)
    out = pl.pallas_call(
        _masked_relu_body,
        grid_spec=pltpu.PrefetchScalarGridSpec(
            num_scalar_prefetch=1,
            grid=(_B, _R // _RB),
            in_specs=[pl.BlockSpec((1, _RB, _C), lambda b, r, m: (b, r, 0))],
            out_specs=pl.BlockSpec((1, _RB, _C), lambda b, r, m: (b, r, 0)),
        ),
        out_shape=jax.ShapeDtypeStruct((_B, _R, _C), x.dtype),
    )(mask, xv)
    return out.reshape(x.shape)
